# Initial kernel scaffold; baseline (speedup 1.0000x reference)
import jax
import jax.numpy as jnp
from jax.experimental import pallas as pl


def kernel(x0, embedding0):
    raise NotImplementedError("write your pallas kernel here")


# speedup vs baseline: 2.0643x; 2.0643x over previous
import functools

import jax
import jax.numpy as jnp
from jax import lax
from jax.experimental import pallas as pl
from jax.experimental.pallas import tpu as pltpu
from jax.experimental.pallas import tpu_sc as plsc

_R = 2048
_K = 1024
_V = 32
_BR = 512
_G = _R // _BR

_GROUPS = [[8 * j + s for s in range(8)] for j in range(4)]


def _fold8(g):
    return ((g[0] + g[4]) + (g[2] + g[6])) + ((g[1] + g[5]) + (g[3] + g[7]))


def _tc_body(x_ref, et_ref, idx_ref, d2_ref, hist_ref, ent_ref):
    i = pl.program_id(0)
    x = x_ref[...]
    et = et_ref[...]
    acc = None
    for grp in _GROUPS:
        planes = []
        for v in grp:
            dv = x[:, v:v + 1] - et[v:v + 1, :]
            planes.append(dv * dv)
        p = _fold8(planes)
        acc = p if acc is None else acc + p
    d = acc * lax.rsqrt(acc)
    iota = lax.broadcasted_iota(jnp.int32, (_BR, _K), 1)
    dmin = jnp.min(d, axis=1, keepdims=True)
    idx = jnp.min(jnp.where(d == dmin, iota, _K), axis=1)
    idx_ref[0, 0, :] = idx
    d2_ref[0, 0, :] = jnp.min(acc, axis=1)
    oh = jnp.where(iota == idx[:, None], 1.0, 0.0).astype(jnp.float32)
    h = jnp.sum(oh, axis=0, keepdims=True)

    @pl.when(i == 0)
    def _init():
        hist_ref[...] = jnp.zeros_like(hist_ref)

    hist_ref[...] += h

    @pl.when(i == _G - 1)
    def _fin():
        hh = hist_ref[...]
        prob = hh * (1.0 / _R)
        safe = jnp.where(hh > 0, prob, 1.0)
        ent_ref[...] = (-jnp.sum(safe * jnp.log(safe)))[None, None]


_tc_call = pl.pallas_call(
    _tc_body,
    grid=(_G,),
    in_specs=[
        pl.BlockSpec((_BR, _V), lambda i: (i, 0)),
        pl.BlockSpec((_V, _K), lambda i: (0, 0)),
    ],
    out_specs=[
        pl.BlockSpec((1, 1, _BR), lambda i: (i, 0, 0)),
        pl.BlockSpec((1, 1, _BR), lambda i: (i, 0, 0)),
        pl.BlockSpec((1, _K), lambda i: (0, 0)),
        pl.BlockSpec((1, 1), lambda i: (0, 0)),
    ],
    out_shape=[
        jax.ShapeDtypeStruct((_G, 1, _BR), jnp.int32),
        jax.ShapeDtypeStruct((_G, 1, _BR), jnp.float32),
        jax.ShapeDtypeStruct((1, _K), jnp.float32),
        jax.ShapeDtypeStruct((1, 1), jnp.float32),
    ],
)


_DP = 128


@functools.cache
def _sc_gather():
    info = plsc.get_sparse_core_info()
    nc, ns = info.num_cores, info.num_subcores
    nw = nc * ns
    bw = _R // nw
    mesh = plsc.VectorSubcoreMesh(core_axis_name="c", subcore_axis_name="s")

    @functools.partial(
        pl.kernel,
        mesh=mesh,
        out_type=jax.ShapeDtypeStruct((_R, _DP), jnp.float32),
        scratch_types=[
            pltpu.VMEM((bw,), jnp.int32),
            pltpu.VMEM((bw, _DP), jnp.float32),
            pltpu.SemaphoreType.DMA,
        ],
    )
    def gather_k(table_hbm, idx_hbm, out_hbm, idx_v, rows_v, sem):
        wid = lax.axis_index("s") * nc + lax.axis_index("c")
        base = wid * bw
        pltpu.sync_copy(idx_hbm.at[pl.ds(base, bw)], idx_v)
        pltpu.async_copy(table_hbm.at[idx_v], rows_v, sem).wait()
        pltpu.sync_copy(rows_v, out_hbm.at[pl.ds(base, bw)])

    return gather_k


def kernel(x0, embedding0):
    x2 = x0.reshape(_R, _V)
    emb = embedding0.reshape(_K, _V)
    idx4, d24, _hist, ent = _tc_call(x2, emb.T)
    idx = idx4.reshape(_R)
    table = jnp.pad(emb, ((0, 0), (0, _DP - _V)))
    out0 = _sc_gather()(table, idx)[:, :_V].reshape(x0.shape)
    out1 = d24.reshape(x0.shape[0], x0.shape[1], x0.shape[2])
    return (out0, out1, out1, ent[0, 0])
